# baseline (speedup 1.0000x reference)
import math

import jax
import jax.numpy as jnp
from jax import lax
from jax.experimental import pallas as pl
from jax.experimental.pallas import tpu as pltpu
from jax.experimental.pallas import tpu_sc as plsc

_VOCAB = 100000
_EMB = 64
_B = 16384
_L = 20

_NC = 2
_NS = 16
_NW = _NC * _NS
_ROWS_PER_W = _B // _NW
_CHUNK = 32
_NCHUNK = _ROWS_PER_W // _CHUNK
_IDX_PER_CHUNK = _CHUNK * _L
_GATHER_N = 128
_NGATHER = _IDX_PER_CHUNK // _GATHER_N


def _sc_pool_body(
    table_hbm, ids_hbm, out_hbm,
    idx_v, rows0, rows1, out0, out1, sem0, sem1, semo0, semo1,
):
    wid = lax.axis_index("s") * _NC + lax.axis_index("c")
    base_row = wid * _ROWS_PER_W
    lane = lax.iota(jnp.int32, 16)
    tail = lane >= 12

    pltpu.sync_copy(
        ids_hbm.at[pl.ds(base_row * _L, _ROWS_PER_W * _L)], idx_v
    )

    bufs = ((rows0, out0, sem0, semo0), (rows1, out1, sem1, semo1))

    def prefetch(c, p):
        rows_v, _, sem, _ = bufs[p]
        for j in range(_NGATHER):
            pltpu.async_copy(
                table_hbm.at[
                    idx_v.at[pl.ds(c * _IDX_PER_CHUNK + j * _GATHER_N, _GATHER_N)]
                ],
                rows_v.at[pl.ds(j * _GATHER_N, _GATHER_N)],
                sem,
            )

    def process(c, p):
        rows_v, out_v, sem, semo = bufs[p]
        pltpu.make_async_copy(
            table_hbm.at[pl.ds(0, _IDX_PER_CHUNK)], rows_v, sem
        ).wait()

        @pl.when(c >= 2)
        def _():
            pltpu.make_async_copy(
                out_hbm.at[pl.ds(0, _CHUNK)], out_v, semo
            ).wait()

        cbase = c * _IDX_PER_CHUNK

        @plsc.parallel_loop(0, _CHUNK, unroll=8)
        def _pool(b):
            ib = b * _L
            v1 = idx_v[pl.ds(cbase + ib, 16)]
            v2 = idx_v[pl.ds(cbase + ib + 4, 16)]
            c1 = plsc.all_reduce_population_count(v1 != 0)
            c2 = plsc.all_reduce_population_count(tail & (v2 != 0))
            cnt = jnp.maximum((c1 + c2).astype(jnp.float32), 1.0)
            inv = 1.0 / cnt
            for k in range(4):
                vs = [rows_v[ib + l, pl.ds(k * 16, 16)] for l in range(_L)]
                while len(vs) > 1:
                    vs = [vs[i] + vs[i + 1] for i in range(0, len(vs) - 1, 2)] + (
                        [vs[-1]] if len(vs) % 2 else []
                    )
                out_v[b, pl.ds(k * 16, 16)] = vs[0] * inv

        pltpu.async_copy(
            out_v, out_hbm.at[pl.ds(base_row + c * _CHUNK, _CHUNK)], semo
        )

    prefetch(0, 0)

    def body(i, carry):
        c0 = i * 2
        prefetch(c0 + 1, 1)
        process(c0, 0)

        @pl.when(i < _NCHUNK // 2 - 1)
        def _():
            prefetch(c0 + 2, 0)

        process(c0 + 1, 1)
        return carry

    lax.fori_loop(0, _NCHUNK // 2, body, 0)
    pltpu.make_async_copy(out_hbm.at[pl.ds(0, _CHUNK)], out0, semo0).wait()
    pltpu.make_async_copy(out_hbm.at[pl.ds(0, _CHUNK)], out1, semo1).wait()


_sc_pool = pl.kernel(
    _sc_pool_body,
    out_type=jax.ShapeDtypeStruct((_B, 128), jnp.float32),
    mesh=plsc.VectorSubcoreMesh(core_axis_name="c", subcore_axis_name="s"),
    compiler_params=pltpu.CompilerParams(
        needs_layout_passes=False, use_tc_tiling_on_sc=False
    ),
    scratch_types=[
        pltpu.VMEM((_ROWS_PER_W * _L,), jnp.int32),
        pltpu.VMEM((_IDX_PER_CHUNK, _EMB), jnp.float32),
        pltpu.VMEM((_IDX_PER_CHUNK, _EMB), jnp.float32),
        pltpu.VMEM((_CHUNK, 128), jnp.float32),
        pltpu.VMEM((_CHUNK, 128), jnp.float32),
        pltpu.SemaphoreType.DMA,
        pltpu.SemaphoreType.DMA,
        pltpu.SemaphoreType.DMA,
        pltpu.SemaphoreType.DMA,
    ],
)

_INV_SQRT2 = 1.0 / math.sqrt(2.0)


def _tc_head_body(x_ref, w_ref, b_ref, g_ref, beta_ref, o_ref):
    x = x_ref[:, :_EMB]
    ht = lax.dot_general(
        w_ref[...], x, (((1,), (1,)), ((), ())),
        preferred_element_type=jnp.float32,
    )
    ht = ht + b_ref[...]
    mu = jnp.mean(ht, axis=0, keepdims=True)
    d = ht - mu
    var = jnp.mean(d * d, axis=0, keepdims=True)
    hn = d * lax.rsqrt(var + 1e-5) * g_ref[...] + beta_ref[...]
    o_ref[...] = 0.5 * hn * (1.0 + lax.erf(hn * _INV_SQRT2))


_TC_BLK = 2048


def _tc_head_t(x, W, bc, gc, betac):
    grid = (_B // _TC_BLK,)
    return pl.pallas_call(
        _tc_head_body,
        grid=grid,
        in_specs=[
            pl.BlockSpec((_TC_BLK, 128), lambda i: (i, 0)),
            pl.BlockSpec((_EMB, _EMB), lambda i: (0, 0)),
            pl.BlockSpec((_EMB, 1), lambda i: (0, 0)),
            pl.BlockSpec((_EMB, 1), lambda i: (0, 0)),
            pl.BlockSpec((_EMB, 1), lambda i: (0, 0)),
        ],
        out_specs=pl.BlockSpec((_EMB, _TC_BLK), lambda i: (0, i)),
        out_shape=jax.ShapeDtypeStruct((_EMB, _B), jnp.float32),
        compiler_params=pltpu.CompilerParams(
            dimension_semantics=("arbitrary",)
        ),
    )(x, W, bc, gc, betac)


def kernel(word_ids, word_emb, W, b, ln_gamma, ln_beta):
    table2 = jnp.pad(word_emb, ((0, 0), (0, 64))).reshape(2 * _VOCAB, _EMB)
    ids = (word_ids.astype(jnp.int32) * 2).reshape(-1)
    mean_emb = _sc_pool(table2, ids)
    out_t = _tc_head_t(
        mean_emb,
        W,
        b.reshape(_EMB, 1),
        ln_gamma.reshape(_EMB, 1),
        ln_beta.reshape(_EMB, 1),
    )
    return out_t.T


# speedup vs baseline: 1.0420x; 1.0420x over previous
import math

import jax
import jax.numpy as jnp
from jax import lax
from jax.experimental import pallas as pl
from jax.experimental.pallas import tpu as pltpu
from jax.experimental.pallas import tpu_sc as plsc

_VOCAB = 100000
_EMB = 64
_B = 16384
_L = 20

_NC = 2
_NS = 16
_NW = _NC * _NS
_ROWS_PER_W = _B // _NW
_CHUNK = 32
_NCHUNK = _ROWS_PER_W // _CHUNK
_IDX_PER_CHUNK = _CHUNK * _L
_GATHER_N = 128
_NGATHER = _IDX_PER_CHUNK // _GATHER_N


def _sc_pool_body(
    table_hbm, ids_hbm, out_hbm,
    idx_v, rows0, rows1, out0, out1, sem0, sem1, semo0, semo1,
):
    wid = lax.axis_index("s") * _NC + lax.axis_index("c")
    base_row = wid * _ROWS_PER_W
    lane = lax.iota(jnp.int32, 16)
    tail = lane >= 12

    pltpu.sync_copy(
        ids_hbm.at[pl.ds(base_row * _L, _ROWS_PER_W * _L)], idx_v
    )

    bufs = ((rows0, out0, sem0, semo0), (rows1, out1, sem1, semo1))

    def prefetch(c, p):
        rows_v, _, sem, _ = bufs[p]
        for j in range(_NGATHER):
            pltpu.async_copy(
                table_hbm.at[
                    idx_v.at[pl.ds(c * _IDX_PER_CHUNK + j * _GATHER_N, _GATHER_N)]
                ],
                rows_v.at[pl.ds(j * _GATHER_N, _GATHER_N)],
                sem,
            )

    def process(c, p):
        rows_v, out_v, sem, semo = bufs[p]
        pltpu.make_async_copy(
            table_hbm.at[pl.ds(0, _IDX_PER_CHUNK)], rows_v, sem
        ).wait()

        @pl.when(c >= 2)
        def _():
            pltpu.make_async_copy(
                out_hbm.at[pl.ds(0, _CHUNK)], out_v, semo
            ).wait()

        cbase = c * _IDX_PER_CHUNK

        @plsc.parallel_loop(0, _CHUNK, unroll=2)
        def _pool(b):
            ib = b * _L
            v1 = idx_v[pl.ds(cbase + ib, 16)]
            v2 = idx_v[pl.ds(cbase + ib + 4, 16)]
            c1 = plsc.all_reduce_population_count(v1 != 0)
            c2 = plsc.all_reduce_population_count(tail & (v2 != 0))
            cnt = jnp.maximum((c1 + c2).astype(jnp.float32), 1.0)
            inv = 1.0 / cnt
            for k in range(4):
                vs = [rows_v[ib + l, pl.ds(k * 16, 16)] for l in range(_L)]
                while len(vs) > 1:
                    vs = [vs[i] + vs[i + 1] for i in range(0, len(vs) - 1, 2)] + (
                        [vs[-1]] if len(vs) % 2 else []
                    )
                out_v[b, pl.ds(k * 16, 16)] = vs[0] * inv

        pltpu.async_copy(
            out_v, out_hbm.at[pl.ds(base_row + c * _CHUNK, _CHUNK)], semo
        )

    prefetch(0, 0)

    def body(i, carry):
        c0 = i * 2
        prefetch(c0 + 1, 1)
        process(c0, 0)

        @pl.when(i < _NCHUNK // 2 - 1)
        def _():
            prefetch(c0 + 2, 0)

        process(c0 + 1, 1)
        return carry

    lax.fori_loop(0, _NCHUNK // 2, body, 0)
    pltpu.make_async_copy(out_hbm.at[pl.ds(0, _CHUNK)], out0, semo0).wait()
    pltpu.make_async_copy(out_hbm.at[pl.ds(0, _CHUNK)], out1, semo1).wait()


_sc_pool = pl.kernel(
    _sc_pool_body,
    out_type=jax.ShapeDtypeStruct((_B, 128), jnp.float32),
    mesh=plsc.VectorSubcoreMesh(core_axis_name="c", subcore_axis_name="s"),
    compiler_params=pltpu.CompilerParams(
        needs_layout_passes=False, use_tc_tiling_on_sc=False
    ),
    scratch_types=[
        pltpu.VMEM((_ROWS_PER_W * _L,), jnp.int32),
        pltpu.VMEM((_IDX_PER_CHUNK, _EMB), jnp.float32),
        pltpu.VMEM((_IDX_PER_CHUNK, _EMB), jnp.float32),
        pltpu.VMEM((_CHUNK, 128), jnp.float32),
        pltpu.VMEM((_CHUNK, 128), jnp.float32),
        pltpu.SemaphoreType.DMA,
        pltpu.SemaphoreType.DMA,
        pltpu.SemaphoreType.DMA,
        pltpu.SemaphoreType.DMA,
    ],
)

_INV_SQRT2 = 1.0 / math.sqrt(2.0)


def _tc_head_body(x_ref, w_ref, b_ref, g_ref, beta_ref, o_ref):
    x = x_ref[:, :_EMB]
    ht = lax.dot_general(
        w_ref[...], x, (((1,), (1,)), ((), ())),
        preferred_element_type=jnp.float32,
    )
    ht = ht + b_ref[...]
    mu = jnp.mean(ht, axis=0, keepdims=True)
    d = ht - mu
    var = jnp.mean(d * d, axis=0, keepdims=True)
    hn = d * lax.rsqrt(var + 1e-5) * g_ref[...] + beta_ref[...]
    o_ref[...] = 0.5 * hn * (1.0 + lax.erf(hn * _INV_SQRT2))


_TC_BLK = 2048


def _tc_head_t(x, W, bc, gc, betac):
    grid = (_B // _TC_BLK,)
    return pl.pallas_call(
        _tc_head_body,
        grid=grid,
        in_specs=[
            pl.BlockSpec((_TC_BLK, 128), lambda i: (i, 0)),
            pl.BlockSpec((_EMB, _EMB), lambda i: (0, 0)),
            pl.BlockSpec((_EMB, 1), lambda i: (0, 0)),
            pl.BlockSpec((_EMB, 1), lambda i: (0, 0)),
            pl.BlockSpec((_EMB, 1), lambda i: (0, 0)),
        ],
        out_specs=pl.BlockSpec((_EMB, _TC_BLK), lambda i: (0, i)),
        out_shape=jax.ShapeDtypeStruct((_EMB, _B), jnp.float32),
        compiler_params=pltpu.CompilerParams(
            dimension_semantics=("arbitrary",)
        ),
    )(x, W, bc, gc, betac)


def kernel(word_ids, word_emb, W, b, ln_gamma, ln_beta):
    table2 = jnp.pad(word_emb, ((0, 0), (0, 64))).reshape(2 * _VOCAB, _EMB)
    ids = (word_ids.astype(jnp.int32) * 2).reshape(-1)
    mean_emb = _sc_pool(table2, ids)
    out_t = _tc_head_t(
        mean_emb,
        W,
        b.reshape(_EMB, 1),
        ln_gamma.reshape(_EMB, 1),
        ln_beta.reshape(_EMB, 1),
    )
    return out_t.T
